# Initial kernel scaffold; baseline (speedup 1.0000x reference)
import jax
import jax.numpy as jnp
from jax.experimental import pallas as pl


def kernel(pd_scores, pd_bboxes, anc_points, gt_labels, gt_bboxes, mask_gt):
    raise NotImplementedError("write your pallas kernel here")


# speedup vs baseline: 59.7738x; 59.7738x over previous
import jax
import jax.numpy as jnp
from jax.experimental import pallas as pl

_TOPK = 10


def _assign_body(scores_ref, pdt_ref, anc_ref, gl_ref, gtb_ref, mg_ref,
                 tl_ref, mp_ref, tg_ref, b0_ref, b1_ref, b2_ref, b3_ref):
    scores = scores_ref[0]
    pdt = pdt_ref[0]
    anc = anc_ref[...]
    gl = gl_ref[0]
    gtb = gtb_ref[0]
    mg = mg_ref[0]

    M = gl.shape[0]
    N = scores.shape[0]
    C = scores.shape[1]

    iota_c = jax.lax.broadcasted_iota(jnp.int32, (M, C), 1)
    onehot = (iota_c == gl).astype(jnp.float32)
    logits = jax.lax.dot_general(
        onehot, scores, (((1,), (1,)), ((), ())),
        precision=jax.lax.Precision.HIGHEST,
        preferred_element_type=jnp.float32)
    sc = jax.nn.sigmoid(logits)

    x1 = pdt[0:1, :]
    y1 = pdt[1:2, :]
    x2 = pdt[2:3, :]
    y2 = pdt[3:4, :]
    gx1 = gtb[:, 0:1]
    gy1 = gtb[:, 1:2]
    gx2 = gtb[:, 2:3]
    gy2 = gtb[:, 3:4]

    iw = jnp.clip(jnp.minimum(x2, gx2) - jnp.maximum(x1, gx1), 0, None)
    ih = jnp.clip(jnp.minimum(y2, gy2) - jnp.maximum(y1, gy1), 0, None)
    inter = iw * ih
    pa = (x2 - x1) * (y2 - y1)
    ga = (gx2 - gx1) * (gy2 - gy1)
    union = pa + ga - inter + 1e-7
    iou = inter / union

    ax = anc[0:1, :]
    ay = anc[1:2, :]
    ing = (ax >= gx1) & (ax <= gx2) & (ay >= gy1) & (ay <= gy2)

    iou2 = iou * iou
    iou6 = iou2 * iou2 * iou2
    align = jnp.where(ing, sc * iou6, 0.0)

    iota_n = jax.lax.broadcasted_iota(jnp.int32, (M, N), 1)
    work = align
    selected = jnp.zeros((M, N), jnp.float32)
    for _ in range(_TOPK):
        rmax = jnp.max(work, axis=1, keepdims=True)
        cand = jnp.where(work == rmax, iota_n, N)
        fi = jnp.min(cand, axis=1, keepdims=True)
        sel = iota_n == fi
        selected = jnp.where(sel, 1.0, selected)
        work = jnp.where(sel, -1.0, work)
    topkm = selected * mg

    cmax = jnp.max(align, axis=0, keepdims=True)
    iota_m = jax.lax.broadcasted_iota(jnp.int32, (M, N), 0)
    mstar = jnp.min(jnp.where(align == cmax, iota_m, M), axis=0,
                    keepdims=True)

    onehot_m = iota_m == mstar
    posv = jnp.sum(jnp.where(onehot_m, topkm, 0.0), axis=0,
                   keepdims=True)
    tg = jnp.where(posv > 0, mstar, 0)

    oh_tg = iota_m == tg
    glf = gl.astype(jnp.float32)
    tl = jnp.sum(jnp.where(oh_tg, glf, 0.0), axis=0, keepdims=True) * posv

    tl_ref[0] = tl
    mp_ref[0] = posv
    tg_ref[0] = tg
    pv = posv[0]
    b0_ref[0] = (jnp.sum(jnp.where(oh_tg, gtb[:, 0:1], 0.0), axis=0) * pv)[None]
    b1_ref[0] = (jnp.sum(jnp.where(oh_tg, gtb[:, 1:2], 0.0), axis=0) * pv)[None]
    b2_ref[0] = (jnp.sum(jnp.where(oh_tg, gtb[:, 2:3], 0.0), axis=0) * pv)[None]
    b3_ref[0] = (jnp.sum(jnp.where(oh_tg, gtb[:, 3:4], 0.0), axis=0) * pv)[None]


def kernel(pd_scores, pd_bboxes, anc_points, gt_labels, gt_bboxes, mask_gt):
    B, N, C = pd_scores.shape
    M = gt_labels.shape[1]

    pdt = jnp.transpose(pd_bboxes, (0, 2, 1))
    anct = jnp.transpose(anc_points, (1, 0))

    out_shape = [
        jax.ShapeDtypeStruct((B, 1, N), jnp.float32),
        jax.ShapeDtypeStruct((B, 1, N), jnp.float32),
        jax.ShapeDtypeStruct((B, 1, N), jnp.int32),
        jax.ShapeDtypeStruct((B, 1, N), jnp.float32),
        jax.ShapeDtypeStruct((B, 1, N), jnp.float32),
        jax.ShapeDtypeStruct((B, 1, N), jnp.float32),
        jax.ShapeDtypeStruct((B, 1, N), jnp.float32),
    ]
    row_spec = pl.BlockSpec((1, 1, N), lambda b: (b, 0, 0))
    outs = pl.pallas_call(
        _assign_body,
        grid=(B,),
        in_specs=[
            pl.BlockSpec((1, N, C), lambda b: (b, 0, 0)),
            pl.BlockSpec((1, 4, N), lambda b: (b, 0, 0)),
            pl.BlockSpec((2, N), lambda b: (0, 0)),
            pl.BlockSpec((1, M, 1), lambda b: (b, 0, 0)),
            pl.BlockSpec((1, M, 4), lambda b: (b, 0, 0)),
            pl.BlockSpec((1, M, 1), lambda b: (b, 0, 0)),
        ],
        out_specs=[row_spec] * 7,
        out_shape=out_shape,
    )(pd_scores, pdt, anct, gt_labels, gt_bboxes, mask_gt)

    tl, mp, tg, b0, b1, b2, b3 = (o[:, 0] for o in outs)
    tb = jnp.stack([b0, b1, b2, b3], axis=-1)
    return (tl, tb, mp, tg)


# speedup vs baseline: 65.8965x; 1.1024x over previous
import jax
import jax.numpy as jnp
from jax.experimental import pallas as pl

_TOPK = 10


def _assign_body(scores_ref, pdt_ref, anc_ref, gl_ref, gtb_ref, mg_ref,
                 glt_ref, gtbt_ref,
                 tl_ref, mp_ref, tg_ref, b0_ref, b1_ref, b2_ref, b3_ref):
    scores = scores_ref[0]
    pdt = pdt_ref[0]
    anc = anc_ref[...]
    glc = gl_ref[0]
    gtb = gtb_ref[0]
    mgc = mg_ref[0]
    glt = glt_ref[0]
    gtbt = gtbt_ref[0]

    M = glc.shape[0]
    N = scores.shape[0]
    C = scores.shape[1]

    iota_c = jax.lax.broadcasted_iota(jnp.int32, (M, C), 1)
    onehot = (iota_c == glc).astype(jnp.float32)
    logits = jax.lax.dot_general(
        onehot, scores, (((1,), (1,)), ((), ())),
        precision=jax.lax.Precision.HIGHEST,
        preferred_element_type=jnp.float32)
    sc = jax.nn.sigmoid(logits)

    x1 = pdt[0:1, :]
    y1 = pdt[1:2, :]
    x2 = pdt[2:3, :]
    y2 = pdt[3:4, :]
    gx1 = gtb[:, 0:1]
    gy1 = gtb[:, 1:2]
    gx2 = gtb[:, 2:3]
    gy2 = gtb[:, 3:4]

    iw = jnp.clip(jnp.minimum(x2, gx2) - jnp.maximum(x1, gx1), 0, None)
    ih = jnp.clip(jnp.minimum(y2, gy2) - jnp.maximum(y1, gy1), 0, None)
    inter = iw * ih
    pa = (x2 - x1) * (y2 - y1)
    ga = (gx2 - gx1) * (gy2 - gy1)
    union = pa + ga - inter + 1e-7
    iou = inter / union

    ax = anc[0:1, :]
    ay = anc[1:2, :]
    ing = (ax >= gx1) & (ax <= gx2) & (ay >= gy1) & (ay <= gy2)

    iou2 = iou * iou
    iou6 = iou2 * iou2 * iou2
    align = jnp.where(ing, sc * iou6, 0.0)

    iota_n = jax.lax.broadcasted_iota(jnp.int32, (M, N), 1)
    work = align
    for _ in range(_TOPK):
        rmax = jnp.max(work, axis=1, keepdims=True)
        fi = jnp.min(jnp.where(work == rmax, iota_n, N), axis=1,
                     keepdims=True)
        work = jnp.where(iota_n == fi, -1.0, work)
    topkm = jnp.where(work < 0, mgc, 0.0)

    cmax = jnp.max(align, axis=0, keepdims=True)
    iota_m0 = jax.lax.broadcasted_iota(jnp.int32, (M, N), 0)
    mstar = jnp.min(jnp.where(align == cmax, iota_m0, M), axis=0,
                    keepdims=True)
    iota_m = iota_m0
    onehot_m = iota_m == mstar
    posv = jnp.sum(jnp.where(onehot_m, topkm, 0.0), axis=0,
                   keepdims=True)
    tg = jnp.where(posv > 0, mstar, 0)

    oh_tg = (iota_m == tg).astype(jnp.float32)
    table = jnp.concatenate([glt.astype(jnp.float32), gtbt], axis=0)
    gathered = jax.lax.dot_general(
        table, oh_tg, (((1,), (0,)), ((), ())),
        precision=jax.lax.Precision.HIGHEST,
        preferred_element_type=jnp.float32)

    tl_ref[0] = gathered[0:1] * posv
    mp_ref[0] = posv
    tg_ref[0] = tg
    b0_ref[0] = gathered[1:2] * posv
    b1_ref[0] = gathered[2:3] * posv
    b2_ref[0] = gathered[3:4] * posv
    b3_ref[0] = gathered[4:5] * posv


def kernel(pd_scores, pd_bboxes, anc_points, gt_labels, gt_bboxes, mask_gt):
    B, N, C = pd_scores.shape
    M = gt_labels.shape[1]

    pdt = jnp.transpose(pd_bboxes, (0, 2, 1))
    anct = jnp.transpose(anc_points, (1, 0))
    gtbt = jnp.transpose(gt_bboxes, (0, 2, 1))
    glt = jnp.transpose(gt_labels, (0, 2, 1))

    out_shape = [
        jax.ShapeDtypeStruct((B, 1, N), jnp.float32),
        jax.ShapeDtypeStruct((B, 1, N), jnp.float32),
        jax.ShapeDtypeStruct((B, 1, N), jnp.int32),
        jax.ShapeDtypeStruct((B, 1, N), jnp.float32),
        jax.ShapeDtypeStruct((B, 1, N), jnp.float32),
        jax.ShapeDtypeStruct((B, 1, N), jnp.float32),
        jax.ShapeDtypeStruct((B, 1, N), jnp.float32),
    ]
    row_spec = pl.BlockSpec((1, 1, N), lambda b: (b, 0, 0))
    outs = pl.pallas_call(
        _assign_body,
        grid=(B,),
        in_specs=[
            pl.BlockSpec((1, N, C), lambda b: (b, 0, 0)),
            pl.BlockSpec((1, 4, N), lambda b: (b, 0, 0)),
            pl.BlockSpec((2, N), lambda b: (0, 0)),
            pl.BlockSpec((1, M, 1), lambda b: (b, 0, 0)),
            pl.BlockSpec((1, M, 4), lambda b: (b, 0, 0)),
            pl.BlockSpec((1, M, 1), lambda b: (b, 0, 0)),
            pl.BlockSpec((1, 1, M), lambda b: (b, 0, 0)),
            pl.BlockSpec((1, 4, M), lambda b: (b, 0, 0)),
        ],
        out_specs=[row_spec] * 7,
        out_shape=out_shape,
    )(pd_scores, pdt, anct, gt_labels, gt_bboxes, mask_gt, glt, gtbt)

    tl, mp, tg, b0, b1, b2, b3 = (o[:, 0] for o in outs)
    tb = jnp.stack([b0, b1, b2, b3], axis=-1)
    return (tl, tb, mp, tg)
